# baseline (speedup 1.0000x reference)
import functools

import jax
import jax.numpy as jnp
from jax import lax
from jax.experimental import pallas as pl
from jax.experimental.pallas import tpu as pltpu
from jax.experimental.pallas import tpu_sc as plsc

D = 128
B = 16384
L = 16

_info = plsc.get_sparse_core_info()
_NC, _NS = _info.num_cores, _info.num_subcores
NW = _NC * _NS
ROWS_PER_W = B // NW
TOT_ROWS = 2 * ROWS_PER_W
CHUNK = 64
NITER = TOT_ROWS // CHUNK
GROUPS = CHUNK // L
NBUF = 4
RBLK = 4


def _make_kernel():
    mesh = plsc.VectorSubcoreMesh(core_axis_name="c", subcore_axis_name="s")

    @functools.partial(
        pl.kernel,
        mesh=mesh,
        compiler_params=pltpu.CompilerParams(needs_layout_passes=False),
        out_type=(
            jax.ShapeDtypeStruct((B,), jnp.float32),
            jax.ShapeDtypeStruct((B,), jnp.float32),
        ),
        scratch_types=[
            pltpu.VMEM((TOT_ROWS,), jnp.int32),
            pltpu.VMEM((TOT_ROWS,), jnp.int32),
            pltpu.VMEM((TOT_ROWS,), jnp.int32),
            pltpu.VMEM((NBUF, CHUNK, D), jnp.float32),
            pltpu.VMEM((NBUF, CHUNK, D), jnp.float32),
            pltpu.VMEM((TOT_ROWS,), jnp.float32),
            pltpu.VMEM_SHARED((1000, D), jnp.float32),
            pltpu.SemaphoreType.DMA((NBUF,)),
            pltpu.SemaphoreType.DMA((NBUF,)),
            pltpu.SemaphoreType.DMA((NBUF,)),
        ],
    )
    def trans_e(pos_h, pos_t, pos_r, neg_h, neg_t, neg_r, ent_emb, rel_emb,
                pos_out, neg_out,
                idxh_v, idxt_v, idxr_v, h_v, t_v, out_v, rel_sh,
                sems_h, sems_t, sems_r):
        wid = lax.axis_index("s") * _NC + lax.axis_index("c")
        base = wid * ROWS_PER_W
        lane_ids = lax.iota(jnp.int32, L)

        idx_copies = [
            pltpu.async_copy(src.at[pl.ds(base, ROWS_PER_W)],
                             dst.at[pl.ds(half, ROWS_PER_W)], sems_h.at[0])
            for src, dst, half in (
                (pos_h, idxh_v, 0), (neg_h, idxh_v, ROWS_PER_W),
                (pos_t, idxt_v, 0), (neg_t, idxt_v, ROWS_PER_W),
                (pos_r, idxr_v, 0), (neg_r, idxr_v, ROWS_PER_W),
            )
        ]
        for cp in idx_copies:
            cp.wait()

        def fire_ht(i, b):
            off = i * CHUNK
            pltpu.async_copy(
                ent_emb.at[idxh_v.at[pl.ds(off, CHUNK)]], h_v.at[b],
                sems_h.at[b])
            pltpu.async_copy(
                ent_emb.at[idxt_v.at[pl.ds(off, CHUNK)]], t_v.at[b],
                sems_t.at[b])

        def fire_radd(i, b):
            off = i * CHUNK
            pltpu.async_copy(
                rel_sh.at[idxr_v.at[pl.ds(off, CHUNK)]], h_v.at[b],
                sems_r.at[b], add=True)

        def wait_h(b):
            pltpu.make_async_copy(
                ent_emb.at[pl.ds(0, CHUNK)], h_v.at[b], sems_h.at[b]).wait()

        def wait_t(b):
            pltpu.make_async_copy(
                ent_emb.at[pl.ds(0, CHUNK)], t_v.at[b], sems_t.at[b]).wait()

        def wait_r(b):
            pltpu.make_async_copy(
                ent_emb.at[pl.ds(0, CHUNK)], h_v.at[b], sems_r.at[b]).wait()

        for b in range(NBUF):
            fire_ht(b, b)

        @pl.when(lax.axis_index("s") == 0)
        def _stage_rel():
            pltpu.sync_copy(rel_emb, rel_sh)

        plsc.subcore_barrier()
        wait_h(0)
        fire_radd(0, 0)

        def iter_body(i, _):
            b = lax.rem(i, NBUF)
            b1 = lax.rem(i + 1, NBUF)

            @pl.when(i + 1 < NITER)
            def _radd_next():
                wait_h(b1)
                fire_radd(i + 1, b1)

            wait_t(b)
            wait_r(b)

            @plsc.parallel_loop(0, GROUPS, unroll=1)
            def group_body(g):
                def block_body(k, total, g=g):
                    for kk in range(RBLK):
                        rr = k * RBLK + kk
                        acc = jnp.zeros((L,), jnp.float32)
                        for j in range(D // L):
                            hr = h_v[b, g * L + rr, pl.ds(j * L, L)]
                            t = t_v[b, g * L + rr, pl.ds(j * L, L)]
                            acc = acc + jnp.abs(hr - t)
                        total = jnp.where(
                            lane_ids == rr, jnp.sum(acc), total)
                    return total

                total = lax.fori_loop(
                    0, L // RBLK, block_body, jnp.zeros((L,), jnp.float32))
                out_v[pl.ds(i * CHUNK + g * L, L)] = total

            @pl.when(i + NBUF < NITER)
            def _fire_next():
                fire_ht(i + NBUF, b)
            return 0

        lax.fori_loop(0, NITER, iter_body, 0)

        cp_pos = pltpu.async_copy(out_v.at[pl.ds(0, ROWS_PER_W)],
                                  pos_out.at[pl.ds(base, ROWS_PER_W)],
                                  sems_h.at[0])
        cp_neg = pltpu.async_copy(out_v.at[pl.ds(ROWS_PER_W, ROWS_PER_W)],
                                  neg_out.at[pl.ds(base, ROWS_PER_W)],
                                  sems_t.at[0])
        cp_pos.wait()
        cp_neg.wait()

    return trans_e


_trans_e = _make_kernel()


@jax.jit
def kernel(pos_h, pos_t, pos_r, neg_h, neg_t, neg_r, ent_emb, rel_emb):
    return _trans_e(pos_h, pos_t, pos_r, neg_h, neg_t, neg_r, ent_emb, rel_emb)


# speedup vs baseline: 1.0071x; 1.0071x over previous
import functools

import jax
import jax.numpy as jnp
from jax import lax
from jax.experimental import pallas as pl
from jax.experimental.pallas import tpu as pltpu
from jax.experimental.pallas import tpu_sc as plsc

D = 128
B = 16384
L = 16

_info = plsc.get_sparse_core_info()
_NC, _NS = _info.num_cores, _info.num_subcores
NW = _NC * _NS
ROWS_PER_W = B // NW
TOT_ROWS = 2 * ROWS_PER_W
CHUNK = 64
NITER = TOT_ROWS // CHUNK
GROUPS = CHUNK // L
NBUF = 4
RBLK = 8


def _make_kernel():
    mesh = plsc.VectorSubcoreMesh(core_axis_name="c", subcore_axis_name="s")

    @functools.partial(
        pl.kernel,
        mesh=mesh,
        compiler_params=pltpu.CompilerParams(needs_layout_passes=False),
        out_type=(
            jax.ShapeDtypeStruct((B,), jnp.float32),
            jax.ShapeDtypeStruct((B,), jnp.float32),
        ),
        scratch_types=[
            pltpu.VMEM((TOT_ROWS,), jnp.int32),
            pltpu.VMEM((TOT_ROWS,), jnp.int32),
            pltpu.VMEM((TOT_ROWS,), jnp.int32),
            pltpu.VMEM((NBUF, CHUNK, D), jnp.float32),
            pltpu.VMEM((NBUF, CHUNK, D), jnp.float32),
            pltpu.VMEM((TOT_ROWS,), jnp.float32),
            pltpu.VMEM_SHARED((1000, D), jnp.float32),
            pltpu.SemaphoreType.DMA((NBUF,)),
            pltpu.SemaphoreType.DMA((NBUF,)),
            pltpu.SemaphoreType.DMA((NBUF,)),
        ],
    )
    def trans_e(pos_h, pos_t, pos_r, neg_h, neg_t, neg_r, ent_emb, rel_emb,
                pos_out, neg_out,
                idxh_v, idxt_v, idxr_v, h_v, t_v, out_v, rel_sh,
                sems_h, sems_t, sems_r):
        wid = lax.axis_index("s") * _NC + lax.axis_index("c")
        base = wid * ROWS_PER_W
        lane_ids = lax.iota(jnp.int32, L)

        idx_copies = [
            pltpu.async_copy(src.at[pl.ds(base, ROWS_PER_W)],
                             dst.at[pl.ds(half, ROWS_PER_W)], sems_h.at[0])
            for src, dst, half in (
                (pos_h, idxh_v, 0), (neg_h, idxh_v, ROWS_PER_W),
                (pos_t, idxt_v, 0), (neg_t, idxt_v, ROWS_PER_W),
                (pos_r, idxr_v, 0), (neg_r, idxr_v, ROWS_PER_W),
            )
        ]
        for cp in idx_copies:
            cp.wait()

        def fire_ht(i, b):
            off = i * CHUNK
            pltpu.async_copy(
                ent_emb.at[idxh_v.at[pl.ds(off, CHUNK)]], h_v.at[b],
                sems_h.at[b])
            pltpu.async_copy(
                ent_emb.at[idxt_v.at[pl.ds(off, CHUNK)]], t_v.at[b],
                sems_t.at[b])

        def fire_radd(i, b):
            off = i * CHUNK
            pltpu.async_copy(
                rel_sh.at[idxr_v.at[pl.ds(off, CHUNK)]], h_v.at[b],
                sems_r.at[b], add=True)

        def wait_h(b):
            pltpu.make_async_copy(
                ent_emb.at[pl.ds(0, CHUNK)], h_v.at[b], sems_h.at[b]).wait()

        def wait_t(b):
            pltpu.make_async_copy(
                ent_emb.at[pl.ds(0, CHUNK)], t_v.at[b], sems_t.at[b]).wait()

        def wait_r(b):
            pltpu.make_async_copy(
                ent_emb.at[pl.ds(0, CHUNK)], h_v.at[b], sems_r.at[b]).wait()

        for b in range(NBUF):
            fire_ht(b, b)

        @pl.when(lax.axis_index("s") == 0)
        def _stage_rel():
            pltpu.sync_copy(rel_emb, rel_sh)

        plsc.subcore_barrier()
        wait_h(0)
        fire_radd(0, 0)

        def iter_body(i, _):
            b = lax.rem(i, NBUF)
            b1 = lax.rem(i + 1, NBUF)

            @pl.when(i + 1 < NITER)
            def _radd_next():
                wait_h(b1)
                fire_radd(i + 1, b1)

            wait_t(b)
            wait_r(b)

            @plsc.parallel_loop(0, GROUPS, unroll=1)
            def group_body(g):
                def block_body(k, total, g=g):
                    for kk in range(RBLK):
                        rr = k * RBLK + kk
                        acc = jnp.zeros((L,), jnp.float32)
                        for j in range(D // L):
                            hr = h_v[b, g * L + rr, pl.ds(j * L, L)]
                            t = t_v[b, g * L + rr, pl.ds(j * L, L)]
                            acc = acc + jnp.abs(hr - t)
                        total = jnp.where(
                            lane_ids == rr, jnp.sum(acc), total)
                    return total

                total = lax.fori_loop(
                    0, L // RBLK, block_body, jnp.zeros((L,), jnp.float32))
                out_v[pl.ds(i * CHUNK + g * L, L)] = total

            @pl.when(i + NBUF < NITER)
            def _fire_next():
                fire_ht(i + NBUF, b)
            return 0

        lax.fori_loop(0, NITER, iter_body, 0)

        cp_pos = pltpu.async_copy(out_v.at[pl.ds(0, ROWS_PER_W)],
                                  pos_out.at[pl.ds(base, ROWS_PER_W)],
                                  sems_h.at[0])
        cp_neg = pltpu.async_copy(out_v.at[pl.ds(ROWS_PER_W, ROWS_PER_W)],
                                  neg_out.at[pl.ds(base, ROWS_PER_W)],
                                  sems_t.at[0])
        cp_pos.wait()
        cp_neg.wait()

    return trans_e


_trans_e = _make_kernel()


@jax.jit
def kernel(pos_h, pos_t, pos_r, neg_h, neg_t, neg_r, ent_emb, rel_emb):
    return _trans_e(pos_h, pos_t, pos_r, neg_h, neg_t, neg_r, ent_emb, rel_emb)


# speedup vs baseline: 1.0084x; 1.0013x over previous
import functools

import jax
import jax.numpy as jnp
from jax import lax
from jax.experimental import pallas as pl
from jax.experimental.pallas import tpu as pltpu
from jax.experimental.pallas import tpu_sc as plsc

D = 128
B = 16384
L = 16

_info = plsc.get_sparse_core_info()
_NC, _NS = _info.num_cores, _info.num_subcores
NW = _NC * _NS
ROWS_PER_W = B // NW
TOT_ROWS = 2 * ROWS_PER_W
CHUNK = 64
NITER = TOT_ROWS // CHUNK
GROUPS = CHUNK // L
NBUF = 6
RBLK = 8


def _make_kernel():
    mesh = plsc.VectorSubcoreMesh(core_axis_name="c", subcore_axis_name="s")

    @functools.partial(
        pl.kernel,
        mesh=mesh,
        compiler_params=pltpu.CompilerParams(needs_layout_passes=False),
        out_type=(
            jax.ShapeDtypeStruct((B,), jnp.float32),
            jax.ShapeDtypeStruct((B,), jnp.float32),
        ),
        scratch_types=[
            pltpu.VMEM((TOT_ROWS,), jnp.int32),
            pltpu.VMEM((TOT_ROWS,), jnp.int32),
            pltpu.VMEM((TOT_ROWS,), jnp.int32),
            pltpu.VMEM((NBUF, CHUNK, D), jnp.float32),
            pltpu.VMEM((NBUF, CHUNK, D), jnp.float32),
            pltpu.VMEM((TOT_ROWS,), jnp.float32),
            pltpu.VMEM_SHARED((1000, D), jnp.float32),
            pltpu.SemaphoreType.DMA((NBUF,)),
            pltpu.SemaphoreType.DMA((NBUF,)),
            pltpu.SemaphoreType.DMA((NBUF,)),
        ],
    )
    def trans_e(pos_h, pos_t, pos_r, neg_h, neg_t, neg_r, ent_emb, rel_emb,
                pos_out, neg_out,
                idxh_v, idxt_v, idxr_v, h_v, t_v, out_v, rel_sh,
                sems_h, sems_t, sems_r):
        wid = lax.axis_index("s") * _NC + lax.axis_index("c")
        base = wid * ROWS_PER_W
        lane_ids = lax.iota(jnp.int32, L)

        idx_copies = [
            pltpu.async_copy(src.at[pl.ds(base, ROWS_PER_W)],
                             dst.at[pl.ds(half, ROWS_PER_W)], sems_h.at[0])
            for src, dst, half in (
                (pos_h, idxh_v, 0), (neg_h, idxh_v, ROWS_PER_W),
                (pos_t, idxt_v, 0), (neg_t, idxt_v, ROWS_PER_W),
                (pos_r, idxr_v, 0), (neg_r, idxr_v, ROWS_PER_W),
            )
        ]
        for cp in idx_copies:
            cp.wait()

        def fire_ht(i, b):
            off = i * CHUNK
            pltpu.async_copy(
                ent_emb.at[idxh_v.at[pl.ds(off, CHUNK)]], h_v.at[b],
                sems_h.at[b])
            pltpu.async_copy(
                ent_emb.at[idxt_v.at[pl.ds(off, CHUNK)]], t_v.at[b],
                sems_t.at[b])

        def fire_radd(i, b):
            off = i * CHUNK
            pltpu.async_copy(
                rel_sh.at[idxr_v.at[pl.ds(off, CHUNK)]], h_v.at[b],
                sems_r.at[b], add=True)

        def wait_h(b):
            pltpu.make_async_copy(
                ent_emb.at[pl.ds(0, CHUNK)], h_v.at[b], sems_h.at[b]).wait()

        def wait_t(b):
            pltpu.make_async_copy(
                ent_emb.at[pl.ds(0, CHUNK)], t_v.at[b], sems_t.at[b]).wait()

        def wait_r(b):
            pltpu.make_async_copy(
                ent_emb.at[pl.ds(0, CHUNK)], h_v.at[b], sems_r.at[b]).wait()

        for b in range(NBUF):
            fire_ht(b, b)

        @pl.when(lax.axis_index("s") == 0)
        def _stage_rel():
            pltpu.sync_copy(rel_emb, rel_sh)

        plsc.subcore_barrier()
        wait_h(0)
        fire_radd(0, 0)

        def iter_body(i, _):
            b = lax.rem(i, NBUF)
            b1 = lax.rem(i + 1, NBUF)

            @pl.when(i + 1 < NITER)
            def _radd_next():
                wait_h(b1)
                fire_radd(i + 1, b1)

            wait_t(b)
            wait_r(b)

            @plsc.parallel_loop(0, GROUPS, unroll=1)
            def group_body(g):
                def block_body(k, total, g=g):
                    for kk in range(RBLK):
                        rr = k * RBLK + kk
                        acc = jnp.zeros((L,), jnp.float32)
                        for j in range(D // L):
                            hr = h_v[b, g * L + rr, pl.ds(j * L, L)]
                            t = t_v[b, g * L + rr, pl.ds(j * L, L)]
                            acc = acc + jnp.abs(hr - t)
                        total = jnp.where(
                            lane_ids == rr, jnp.sum(acc), total)
                    return total

                total = lax.fori_loop(
                    0, L // RBLK, block_body, jnp.zeros((L,), jnp.float32))
                out_v[pl.ds(i * CHUNK + g * L, L)] = total

            @pl.when(i + NBUF < NITER)
            def _fire_next():
                fire_ht(i + NBUF, b)
            return 0

        lax.fori_loop(0, NITER, iter_body, 0)

        cp_pos = pltpu.async_copy(out_v.at[pl.ds(0, ROWS_PER_W)],
                                  pos_out.at[pl.ds(base, ROWS_PER_W)],
                                  sems_h.at[0])
        cp_neg = pltpu.async_copy(out_v.at[pl.ds(ROWS_PER_W, ROWS_PER_W)],
                                  neg_out.at[pl.ds(base, ROWS_PER_W)],
                                  sems_t.at[0])
        cp_pos.wait()
        cp_neg.wait()

    return trans_e


_trans_e = _make_kernel()


@jax.jit
def kernel(pos_h, pos_t, pos_r, neg_h, neg_t, neg_r, ent_emb, rel_emb):
    return _trans_e(pos_h, pos_t, pos_r, neg_h, neg_t, neg_r, ent_emb, rel_emb)
